# baseline (speedup 1.0000x reference)
import functools

import jax
import jax.numpy as jnp
from jax.experimental import pallas as pl

_K = 64


def _search16(data, count_fn, k_ref):
    rows = data.shape[0]
    lo0 = jnp.full((rows, 1), -32768, jnp.int32)
    hi0 = jnp.full((rows, 1), 32767, jnp.int32)

    def body(_, carry):
        lo, hi = carry
        mid = (lo + hi + 1) >> 1
        cnt = count_fn(mid)
        ge = cnt >= k_ref
        return jnp.where(ge, mid, lo), jnp.where(ge, hi, mid - 1)

    t, _ = jax.lax.fori_loop(0, 16, body, (lo0, hi0))
    return t


def _topk_mask_kernel(x_ref, o_ref, *, k):
    x = x_ref[...]
    b = jax.lax.bitcast_convert_type(x, jnp.int32)
    key = jnp.where(b >= 0, b, b ^ jnp.int32(0x7FFFFFFF))
    khi = (key >> 16).astype(jnp.int16)
    klo = ((key ^ 0x8000) << 16 >> 16).astype(jnp.int16)

    def cnt_hi(m):
        return jnp.sum((khi >= m.astype(jnp.int16)).astype(jnp.int16),
                       axis=-1, dtype=jnp.int32, keepdims=True)

    p = _search16(khi, cnt_hi, k)
    p16 = p.astype(jnp.int16)
    eq_hi = khi == p16
    cand_lo = jnp.where(eq_hi, klo, jnp.int16(-32768))
    c_gt_hi = jnp.sum((khi > p16).astype(jnp.int16), axis=-1,
                      dtype=jnp.int32, keepdims=True)
    slots_hi = k - c_gt_hi

    def cnt_lo(m):
        return jnp.sum((cand_lo >= m.astype(jnp.int16)).astype(jnp.int16),
                       axis=-1, dtype=jnp.int32, keepdims=True)

    q = _search16(cand_lo, cnt_lo, slots_hi)
    q16 = q.astype(jnp.int16)

    gt = (khi > p16) | (eq_hi & (klo > q16))
    eq = eq_hi & (klo == q16)
    c_eq = jnp.sum(eq.astype(jnp.int16), axis=-1, dtype=jnp.int32,
                   keepdims=True)
    c_ge = (k - slots_hi) + jnp.sum((eq_hi & (klo >= q16)).astype(jnp.int16),
                                    axis=-1, dtype=jnp.int32, keepdims=True)
    relu = jnp.maximum(x, 0.0)
    need_fix = jnp.any((c_ge > k) & (p >= 0))

    @pl.when(jnp.logical_not(need_fix))
    def _():
        o_ref[...] = jnp.where(gt | eq, relu, 0.0)

    @pl.when(need_fix)
    def _():
        slots = k - (c_ge - c_eq)
        rows = x.shape[0]
        idx = jax.lax.broadcasted_iota(jnp.int32, x.shape, 1)
        lo2 = jnp.zeros((rows, 1), jnp.int32)
        hi2 = jnp.full((rows, 1), x.shape[1] - 1, jnp.int32)

        def body2(_, carry):
            l, h = carry
            m = (l + h) >> 1
            c = jnp.sum((eq & (idx <= m)).astype(jnp.int32), axis=-1,
                        keepdims=True)
            enough = c >= slots
            return jnp.where(enough, l, m + 1), jnp.where(enough, m, h)

        cut, _ = jax.lax.fori_loop(0, 15, body2, (lo2, hi2))
        keep = gt | (eq & (idx <= cut))
        o_ref[...] = jnp.where(keep, relu, 0.0)


def kernel(features):
    B, L, D = features.shape
    x = features.reshape(B * L, D)
    rows_per_block = next(r for r in (32, 16, 8, 4, 2, 1) if (B * L) % r == 0)
    out = pl.pallas_call(
        functools.partial(_topk_mask_kernel, k=_K),
        grid=((B * L) // rows_per_block,),
        in_specs=[pl.BlockSpec((rows_per_block, D), lambda i: (i, 0))],
        out_specs=pl.BlockSpec((rows_per_block, D), lambda i: (i, 0)),
        out_shape=jax.ShapeDtypeStruct((B * L, D), jnp.float32),
    )(x)
    return out.reshape(B, L, D)
